# baseline (speedup 1.0000x reference)
import functools

import numpy as np

import jax
import jax.numpy as jnp
from jax import lax
from jax.experimental import pallas as pl
from jax.experimental.pallas import tpu as pltpu
from jax.experimental.pallas import tpu_sc as plsc

_TOPK = 2048
_N = 32768
_ROWS = 64
_NC = 2
_NS = 16
_NW = _NC * _NS
_SC_ROWS = 32
_TC_ROWS = _ROWS - _SC_ROWS
_L = 16
_UNROLL = 8
_NQ = 8
_QE = _N // _NQ

_SIGN = np.uint32(0x80000000)


def _to_key(x):
    u = plsc.bitcast(x, jnp.uint32)
    neg = u >= _SIGN
    return jnp.where(neg, ~u, u | _SIGN)


def _key_val(ku):
    pos = ku >= _SIGN
    return plsc.bitcast(jnp.where(pos, ku ^ _SIGN, ~ku), jnp.float32)


def _digit(ku, sh):
    return (lax.shift_right_logical(ku, jnp.uint32(sh))
            & jnp.uint32(15)).astype(jnp.int32)


_RPW = _SC_ROWS // _NW


@functools.partial(
    pl.kernel,
    out_type=jax.ShapeDtypeStruct((_NW, _L), jnp.float32),
    mesh=plsc.VectorSubcoreMesh(
        core_axis_name="c", subcore_axis_name="s",
        num_cores=_NC, num_subcores=_NS),
    compiler_params=pltpu.CompilerParams(needs_layout_passes=False),
    scratch_types=[
        pltpu.VMEM((_N,), jnp.float32),
        pltpu.VMEM((_N,), jnp.int32),
        pltpu.VMEM((_N,), jnp.int32),
        pltpu.VMEM((_L, _L), jnp.int32),
        pltpu.VMEM((_L,), jnp.float32),
        [pltpu.SemaphoreType.DMA] * _NQ,
    ],
)
def _sc_topk(scores_hbm, out_hbm, rowbuf, bufa, bufb, hist, outv, sems):
    iota = lax.iota(jnp.int32, _L)
    ones = jnp.ones((_L,), jnp.int32)
    zeros16i = jnp.zeros((_L,), jnp.int32)
    zeros16f = jnp.zeros((_L,), jnp.float32)
    wid = lax.axis_index("s") * _NC + lax.axis_index("c")
    outv[...] = zeros16f

    def row_copy(r, q):
        return pltpu.make_async_copy(
            scores_hbm.at[r, pl.ds(q * _QE, _QE)],
            rowbuf.at[pl.ds(q * _QE, _QE)],
            sems[q])

    def clear_hist():
        for j in range(_L):
            hist[j] = zeros16i

    def select(k_rem):
        totals = hist[0]
        for j in range(1, _L):
            unskew = (iota + j) & 15
            totals = totals + hist[j].at[unskew].get(
                mode="promise_in_bounds")
        cge = jnp.flip(jnp.cumsum(jnp.flip(totals)))
        pc = plsc.all_reduce_population_count(cge >= k_rem)
        b = jnp.max(pc) - 1
        cnt_gt = jnp.sum(jnp.where(iota > b, totals, 0))
        return b, cnt_gt

    for q in range(_NQ):
        row_copy(wid * _RPW, q).start()

    for i in range(_RPW):
        row = wid * _RPW + i

        clear_hist()
        for q in range(_NQ):
            row_copy(row, q).wait()

            def h0(ci, q=q):
                ku = _to_key(rowbuf[pl.ds((q * _QE // _L + ci) * _L, _L)])
                col = (_digit(ku, 28) + iota) & 15
                plsc.addupdate_scatter(hist, [iota, col], ones)

            plsc.parallel_loop(0, _QE // _L, unroll=_UNROLL)(h0)

        b, cnt_gt = select(jnp.int32(_TOPK))
        k_rem = jnp.int32(_TOPK) - cnt_gt
        t_bits = b << 28

        clear_hist()

        def c0(ci, carry):
            cntv, sacc = carry
            xv = rowbuf[pl.ds(ci * _L, _L)]
            ku = _to_key(xv)
            digit = _digit(ku, 28)
            sacc = sacc + jnp.where(digit > b, xv, jnp.float32(0.0))
            eqm = digit == b
            plsc.store_scatter(bufa, [cntv * _L + iota],
                               plsc.bitcast(ku, jnp.int32), mask=eqm)
            col = (_digit(ku, 24) + iota) & 15
            plsc.addupdate_scatter(hist, [iota, col], ones, mask=eqm)
            return cntv + eqm.astype(jnp.int32), sacc

        cntv, sacc = plsc.parallel_loop(
            0, _N // _L, unroll=_UNROLL, carry=(zeros16i, zeros16f))(c0)

        if i + 1 < _RPW:
            for q in range(_NQ):
                row_copy(row + 1, q).start()

        src, dst = bufa, bufb
        for l in range(1, 8):
            sh = 28 - 4 * l
            b, cnt_gt = select(k_rem)
            t_bits = t_bits | (b << sh)
            k_rem = k_rem - cnt_gt

            if l < 7:
                nch = jnp.max(cntv)
                clear_hist()

                def cl(ci, carry, src=src, dst=dst, sh=sh, cntv=cntv, b=b):
                    ncntv, sacc = carry
                    kv = src[pl.ds(ci * _L, _L)]
                    ku = plsc.bitcast(kv, jnp.uint32)
                    digit = _digit(ku, sh)
                    valid = ci < cntv
                    gtm = valid & (digit > b)
                    sacc = sacc + jnp.where(gtm, _key_val(ku),
                                            jnp.float32(0.0))
                    eqm = valid & (digit == b)
                    plsc.store_scatter(dst, [ncntv * _L + iota],
                                       kv, mask=eqm)
                    col = (_digit(ku, sh - 4) + iota) & 15
                    plsc.addupdate_scatter(hist, [iota, col],
                                           ones, mask=eqm)
                    return ncntv + eqm.astype(jnp.int32), sacc

                cntv, sacc = plsc.parallel_loop(
                    0, nch, unroll=_UNROLL, carry=(zeros16i, sacc))(cl)
                src, dst = dst, src

        t_vec = _key_val(plsc.bitcast(jnp.full((_L,), t_bits, jnp.int32),
                                      jnp.uint32))
        mean_vec = (jnp.sum(sacc) + k_rem.astype(jnp.float32) * t_vec) \
            * jnp.float32(1.0 / _TOPK)
        outv[...] = jnp.where(iota == i, mean_vec, outv[...])

    pltpu.sync_copy(outv, out_hbm.at[wid])


def _tc_topk_body(x_ref, o_ref):
    x = x_ref[0]
    u = lax.bitcast_convert_type(x, jnp.uint32)
    neg = u >= jnp.uint32(0x80000000)
    su = jnp.where(neg, ~u, u | jnp.uint32(0x80000000))

    def step(i, prefix):
        bit = jnp.uint32(1) << (jnp.uint32(31) - i.astype(jnp.uint32))
        cand = prefix | bit
        ge = su >= cand[:, None, None]
        cnt = jnp.sum(ge.astype(jnp.int32), axis=(1, 2))
        return jnp.where(cnt >= _TOPK, cand, prefix)

    t = lax.fori_loop(0, 32, step, jnp.zeros((_TC_ROWS,), jnp.uint32))
    gt = su > t[:, None, None]
    cnt_gt = jnp.sum(gt.astype(jnp.int32), axis=(1, 2))
    sum_gt = jnp.sum(jnp.where(gt, x, 0.0), axis=(1, 2))
    tv_u = jnp.where(t >= jnp.uint32(0x80000000),
                     t ^ jnp.uint32(0x80000000), ~t)
    tval = lax.bitcast_convert_type(tv_u, jnp.float32)
    total = sum_gt + (jnp.float32(_TOPK) - cnt_gt.astype(jnp.float32)) * tval
    mean = total * jnp.float32(1.0 / _TOPK)
    o_ref[...] = jnp.broadcast_to(mean[:, None], (_TC_ROWS, 128))


def kernel(scores):
    sc_out = _sc_topk(scores)
    tc_out = pl.pallas_call(
        _tc_topk_body,
        out_shape=jax.ShapeDtypeStruct((_TC_ROWS, 128), jnp.float32),
        grid=(1,),
        in_specs=[pl.BlockSpec((1, _TC_ROWS, 256, 128),
                               lambda i: (1, 0, 0, 0))],
        out_specs=pl.BlockSpec((_TC_ROWS, 128), lambda i: (0, 0)),
    )(scores.reshape(2, _TC_ROWS, 256, 128))
    sc_means = sc_out[:, :_RPW].reshape(-1)
    return jnp.concatenate([sc_means, tc_out[:, 0]])


# speedup vs baseline: 1.2163x; 1.2163x over previous
import functools

import numpy as np

import jax
import jax.numpy as jnp
from jax import lax
from jax.experimental import pallas as pl
from jax.experimental.pallas import tpu as pltpu
from jax.experimental.pallas import tpu_sc as plsc

_TOPK = 2048
_N = 32768
_ROWS = 64
_NC = 2
_NS = 16
_NW = _NC * _NS
_SC_ROWS = 32
_TC_ROWS = _ROWS - _SC_ROWS
_L = 16
_UNROLL = 8
_NQ = 8
_QE = _N // _NQ

_SIGN = np.uint32(0x80000000)


def _to_key(x):
    u = plsc.bitcast(x, jnp.uint32)
    neg = u >= _SIGN
    return jnp.where(neg, ~u, u | _SIGN)


def _key_val(ku):
    pos = ku >= _SIGN
    return plsc.bitcast(jnp.where(pos, ku ^ _SIGN, ~ku), jnp.float32)


def _digit(ku, sh):
    return (lax.shift_right_logical(ku, jnp.uint32(sh))
            & jnp.uint32(15)).astype(jnp.int32)


_RPW = _SC_ROWS // _NW


@functools.partial(
    pl.kernel,
    out_type=jax.ShapeDtypeStruct((_NW, _L), jnp.float32),
    mesh=plsc.VectorSubcoreMesh(
        core_axis_name="c", subcore_axis_name="s",
        num_cores=_NC, num_subcores=_NS),
    compiler_params=pltpu.CompilerParams(needs_layout_passes=False),
    scratch_types=[
        pltpu.VMEM((_N,), jnp.float32),
        pltpu.VMEM((_N,), jnp.int32),
        pltpu.VMEM((_N,), jnp.int32),
        pltpu.VMEM((_L, _L), jnp.int32),
        pltpu.VMEM((_L,), jnp.float32),
        [pltpu.SemaphoreType.DMA] * _NQ,
    ],
)
def _sc_topk(scores_hbm, out_hbm, rowbuf, bufa, bufb, hist, outv, sems):
    iota = lax.iota(jnp.int32, _L)
    ones = jnp.ones((_L,), jnp.int32)
    zeros16i = jnp.zeros((_L,), jnp.int32)
    zeros16f = jnp.zeros((_L,), jnp.float32)
    wid = lax.axis_index("s") * _NC + lax.axis_index("c")
    outv[...] = zeros16f

    def row_copy(r, q):
        return pltpu.make_async_copy(
            scores_hbm.at[r, pl.ds(q * _QE, _QE)],
            rowbuf.at[pl.ds(q * _QE, _QE)],
            sems[q])

    def clear_hist():
        for j in range(_L):
            hist[j] = zeros16i

    def select(k_rem):
        totals = hist[0]
        for j in range(1, _L):
            unskew = (iota + j) & 15
            totals = totals + hist[j].at[unskew].get(
                mode="promise_in_bounds")
        cge = jnp.flip(jnp.cumsum(jnp.flip(totals)))
        pc = plsc.all_reduce_population_count(cge >= k_rem)
        b = jnp.max(pc) - 1
        cnt_gt = jnp.sum(jnp.where(iota > b, totals, 0))
        return b, cnt_gt

    for q in range(_NQ):
        row_copy(wid * _RPW, q).start()

    for i in range(_RPW):
        row = wid * _RPW + i

        clear_hist()
        for q in range(_NQ):
            row_copy(row, q).wait()

            def h0(ci, q=q):
                ku = _to_key(rowbuf[pl.ds((q * _QE // _L + ci) * _L, _L)])
                col = (_digit(ku, 28) + iota) & 15
                plsc.addupdate_scatter(hist, [iota, col], ones)

            plsc.parallel_loop(0, _QE // _L, unroll=_UNROLL)(h0)

        b, cnt_gt = select(jnp.int32(_TOPK))
        k_rem = jnp.int32(_TOPK) - cnt_gt
        t_bits = b << 28

        clear_hist()

        def c0(ci, carry):
            cntv, sacc = carry
            xv = rowbuf[pl.ds(ci * _L, _L)]
            ku = _to_key(xv)
            digit = _digit(ku, 28)
            sacc = sacc + jnp.where(digit > b, xv, jnp.float32(0.0))
            eqm = digit == b
            plsc.store_scatter(bufa, [cntv * _L + iota],
                               plsc.bitcast(ku, jnp.int32), mask=eqm)
            col = (_digit(ku, 24) + iota) & 15
            plsc.addupdate_scatter(hist, [iota, col], ones, mask=eqm)
            return cntv + eqm.astype(jnp.int32), sacc

        cntv, sacc = plsc.parallel_loop(
            0, _N // _L, unroll=_UNROLL, carry=(zeros16i, zeros16f))(c0)

        if i + 1 < _RPW:
            for q in range(_NQ):
                row_copy(row + 1, q).start()

        src, dst = bufa, bufb
        for l in range(1, 8):
            sh = 28 - 4 * l
            b, cnt_gt = select(k_rem)
            t_bits = t_bits | (b << sh)
            k_rem = k_rem - cnt_gt

            if l < 7:
                nch = jnp.max(cntv)
                clear_hist()

                def cl(ci, carry, src=src, dst=dst, sh=sh, cntv=cntv, b=b):
                    ncntv, sacc = carry
                    kv = src[pl.ds(ci * _L, _L)]
                    ku = plsc.bitcast(kv, jnp.uint32)
                    digit = _digit(ku, sh)
                    valid = ci < cntv
                    gtm = valid & (digit > b)
                    sacc = sacc + jnp.where(gtm, _key_val(ku),
                                            jnp.float32(0.0))
                    eqm = valid & (digit == b)
                    plsc.store_scatter(dst, [ncntv * _L + iota],
                                       kv, mask=eqm)
                    col = (_digit(ku, sh - 4) + iota) & 15
                    plsc.addupdate_scatter(hist, [iota, col],
                                           ones, mask=eqm)
                    return ncntv + eqm.astype(jnp.int32), sacc

                cntv, sacc = plsc.parallel_loop(
                    0, nch, unroll=_UNROLL, carry=(zeros16i, sacc))(cl)
                src, dst = dst, src

        t_vec = _key_val(plsc.bitcast(jnp.full((_L,), t_bits, jnp.int32),
                                      jnp.uint32))
        mean_vec = (jnp.sum(sacc) + k_rem.astype(jnp.float32) * t_vec) \
            * jnp.float32(1.0 / _TOPK)
        outv[...] = jnp.where(iota == i, mean_vec, outv[...])

    pltpu.sync_copy(outv, out_hbm.at[wid])


def _tc_topk_body(x_ref, o_ref):
    x = x_ref[...].reshape(_TC_ROWS, 256, 128)
    u = lax.bitcast_convert_type(x, jnp.uint32)
    neg = u >= jnp.uint32(0x80000000)
    su = jnp.where(neg, ~u, u | jnp.uint32(0x80000000))

    def step(i, prefix):
        bit = jnp.uint32(1) << (jnp.uint32(31) - i.astype(jnp.uint32))
        cand = prefix | bit
        ge = su >= cand[:, None, None]
        cnt = jnp.sum(ge.astype(jnp.int32), axis=(1, 2))
        return jnp.where(cnt >= _TOPK, cand, prefix)

    t = lax.fori_loop(0, 32, step, jnp.zeros((_TC_ROWS,), jnp.uint32))
    gt = su > t[:, None, None]
    cnt_gt = jnp.sum(gt.astype(jnp.int32), axis=(1, 2))
    sum_gt = jnp.sum(jnp.where(gt, x, 0.0), axis=(1, 2))
    tv_u = jnp.where(t >= jnp.uint32(0x80000000),
                     t ^ jnp.uint32(0x80000000), ~t)
    tval = lax.bitcast_convert_type(tv_u, jnp.float32)
    total = sum_gt + (jnp.float32(_TOPK) - cnt_gt.astype(jnp.float32)) * tval
    mean = total * jnp.float32(1.0 / _TOPK)
    o_ref[...] = jnp.broadcast_to(mean[:, None], (_TC_ROWS, 128))


def kernel(scores):
    sc_out = _sc_topk(scores)
    tc_out = pl.pallas_call(
        _tc_topk_body,
        out_shape=jax.ShapeDtypeStruct((_TC_ROWS, 128), jnp.float32),
        grid=(1,),
        in_specs=[pl.BlockSpec((_TC_ROWS, _N), lambda i: (1, 0))],
        out_specs=pl.BlockSpec((_TC_ROWS, 128), lambda i: (0, 0)),
    )(scores)
    sc_means = sc_out[:, :_RPW].reshape(-1)
    return jnp.concatenate([sc_means, tc_out[:, 0]])


# speedup vs baseline: 1.2282x; 1.0098x over previous
import functools

import numpy as np

import jax
import jax.numpy as jnp
from jax import lax
from jax.experimental import pallas as pl
from jax.experimental.pallas import tpu as pltpu
from jax.experimental.pallas import tpu_sc as plsc

_TOPK = 2048
_N = 32768
_ROWS = 64
_NC = 2
_NS = 16
_NW = _NC * _NS
_SC_ROWS = 32
_TC_ROWS = _ROWS - _SC_ROWS
_L = 16
_UNROLL = 8
_NQ = 8
_QE = _N // _NQ

_SIGN = np.uint32(0x80000000)


def _to_key(x):
    u = plsc.bitcast(x, jnp.uint32)
    neg = u >= _SIGN
    return jnp.where(neg, ~u, u | _SIGN)


def _key_val(ku):
    pos = ku >= _SIGN
    return plsc.bitcast(jnp.where(pos, ku ^ _SIGN, ~ku), jnp.float32)


def _digit(ku, sh):
    return (lax.shift_right_logical(ku, jnp.uint32(sh))
            & jnp.uint32(15)).astype(jnp.int32)


_RPW = _SC_ROWS // _NW


@functools.partial(
    pl.kernel,
    out_type=jax.ShapeDtypeStruct((_NW, _L), jnp.float32),
    mesh=plsc.VectorSubcoreMesh(
        core_axis_name="c", subcore_axis_name="s",
        num_cores=_NC, num_subcores=_NS),
    compiler_params=pltpu.CompilerParams(needs_layout_passes=False),
    scratch_types=[
        pltpu.VMEM((_N,), jnp.float32),
        pltpu.VMEM((_N,), jnp.int32),
        pltpu.VMEM((_N,), jnp.int32),
        pltpu.VMEM((_L, 256), jnp.int32),
        pltpu.VMEM((_L, _L), jnp.int32),
        pltpu.VMEM((_L,), jnp.float32),
        [pltpu.SemaphoreType.DMA] * _NQ,
    ],
)
def _sc_topk(scores_hbm, out_hbm, rowbuf, bufa, bufb, hist8, hist, outv,
             sems):
    iota = lax.iota(jnp.int32, _L)
    ones = jnp.ones((_L,), jnp.int32)
    zeros16i = jnp.zeros((_L,), jnp.int32)
    zeros16f = jnp.zeros((_L,), jnp.float32)
    wid = lax.axis_index("s") * _NC + lax.axis_index("c")
    outv[...] = zeros16f

    def row_copy(r, q):
        return pltpu.make_async_copy(
            scores_hbm.at[r, pl.ds(q * _QE, _QE)],
            rowbuf.at[pl.ds(q * _QE, _QE)],
            sems[q])

    def clear_hist():
        for j in range(_L):
            hist[j] = zeros16i

    def select(k_rem):
        totals = hist[0]
        for j in range(1, _L):
            unskew = (iota + j) & 15
            totals = totals + hist[j].at[unskew].get(
                mode="promise_in_bounds")
        cge = jnp.flip(jnp.cumsum(jnp.flip(totals)))
        pc = plsc.all_reduce_population_count(cge >= k_rem)
        b = jnp.max(pc) - 1
        cnt_gt = jnp.sum(jnp.where(iota > b, totals, 0))
        return b, cnt_gt

    def select8(k_rem):
        zero_idx = jnp.zeros((_L,), jnp.int32)
        totals = []
        for c in range(16):
            acc = zeros16i
            for j in range(_L):
                idx = (iota + (16 * c + j)) & 255
                acc = acc + plsc.load_gather(
                    hist8, [jnp.full((_L,), j, jnp.int32), idx])
            totals.append(acc)
        run = zeros16i
        cge = [None] * 16
        for c in range(15, -1, -1):
            s = jnp.flip(jnp.cumsum(jnp.flip(totals[c]))) + run
            cge[c] = s
            run = s.at[zero_idx].get(mode="promise_in_bounds")
        pcv = zeros16i
        for c in range(16):
            pcv = pcv + plsc.all_reduce_population_count(cge[c] >= k_rem)
        b8 = jnp.max(pcv) - 1
        accg = zeros16i
        for c in range(16):
            accg = accg + jnp.where(iota + 16 * c > b8, totals[c], 0)
        cnt_gt = jnp.sum(accg)
        return b8, cnt_gt

    for q in range(_NQ):
        row_copy(wid * _RPW, q).start()

    for i in range(_RPW):
        row = wid * _RPW + i

        for j in range(_L):
            for c in range(16):
                hist8[j, pl.ds(c * _L, _L)] = zeros16i
        for q in range(_NQ):
            row_copy(row, q).wait()

            def h0(ci, q=q):
                ku = _to_key(rowbuf[pl.ds((q * _QE // _L + ci) * _L, _L)])
                d8 = lax.shift_right_logical(
                    ku, jnp.uint32(24)).astype(jnp.int32)
                col = (d8 + iota) & 255
                plsc.addupdate_scatter(hist8, [iota, col], ones)

            plsc.parallel_loop(0, _QE // _L, unroll=_UNROLL)(h0)

        b, cnt_gt = select8(jnp.int32(_TOPK))
        k_rem = jnp.int32(_TOPK) - cnt_gt
        t_bits = b << 24

        clear_hist()

        def c0(ci, carry):
            cntv, sacc = carry
            xv = rowbuf[pl.ds(ci * _L, _L)]
            ku = _to_key(xv)
            d8 = lax.shift_right_logical(
                ku, jnp.uint32(24)).astype(jnp.int32)
            sacc = sacc + jnp.where(d8 > b, xv, jnp.float32(0.0))
            eqm = d8 == b
            plsc.store_scatter(bufa, [cntv * _L + iota],
                               plsc.bitcast(ku, jnp.int32), mask=eqm)
            col = (_digit(ku, 20) + iota) & 15
            plsc.addupdate_scatter(hist, [iota, col], ones, mask=eqm)
            return cntv + eqm.astype(jnp.int32), sacc

        cntv, sacc = plsc.parallel_loop(
            0, _N // _L, unroll=_UNROLL, carry=(zeros16i, zeros16f))(c0)

        if i + 1 < _RPW:
            for q in range(_NQ):
                row_copy(row + 1, q).start()

        src, dst = bufa, bufb
        for sh in (20, 16, 12, 8, 4, 0):
            b, cnt_gt = select(k_rem)
            t_bits = t_bits | (b << sh)
            k_rem = k_rem - cnt_gt

            if sh > 0:
                nch = jnp.max(cntv)
                clear_hist()

                def cl(ci, carry, src=src, dst=dst, sh=sh, cntv=cntv, b=b):
                    ncntv, sacc = carry
                    kv = src[pl.ds(ci * _L, _L)]
                    ku = plsc.bitcast(kv, jnp.uint32)
                    digit = _digit(ku, sh)
                    valid = ci < cntv
                    gtm = valid & (digit > b)
                    sacc = sacc + jnp.where(gtm, _key_val(ku),
                                            jnp.float32(0.0))
                    eqm = valid & (digit == b)
                    plsc.store_scatter(dst, [ncntv * _L + iota],
                                       kv, mask=eqm)
                    col = (_digit(ku, sh - 4) + iota) & 15
                    plsc.addupdate_scatter(hist, [iota, col],
                                           ones, mask=eqm)
                    return ncntv + eqm.astype(jnp.int32), sacc

                cntv, sacc = plsc.parallel_loop(
                    0, nch, unroll=_UNROLL, carry=(zeros16i, sacc))(cl)
                src, dst = dst, src

        t_vec = _key_val(plsc.bitcast(jnp.full((_L,), t_bits, jnp.int32),
                                      jnp.uint32))
        mean_vec = (jnp.sum(sacc) + k_rem.astype(jnp.float32) * t_vec) \
            * jnp.float32(1.0 / _TOPK)
        outv[...] = jnp.where(iota == i, mean_vec, outv[...])

    pltpu.sync_copy(outv, out_hbm.at[wid])


def _tc_topk_body(x_ref, o_ref):
    x = x_ref[...].reshape(_TC_ROWS, 256, 128)
    u = lax.bitcast_convert_type(x, jnp.uint32)
    neg = u >= jnp.uint32(0x80000000)
    su = jnp.where(neg, ~u, u | jnp.uint32(0x80000000))

    def step(i, prefix):
        bit = jnp.uint32(1) << (jnp.uint32(31) - i.astype(jnp.uint32))
        cand = prefix | bit
        ge = su >= cand[:, None, None]
        cnt = jnp.sum(ge.astype(jnp.int32), axis=(1, 2))
        return jnp.where(cnt >= _TOPK, cand, prefix)

    t = lax.fori_loop(0, 32, step, jnp.zeros((_TC_ROWS,), jnp.uint32))
    gt = su > t[:, None, None]
    cnt_gt = jnp.sum(gt.astype(jnp.int32), axis=(1, 2))
    sum_gt = jnp.sum(jnp.where(gt, x, 0.0), axis=(1, 2))
    tv_u = jnp.where(t >= jnp.uint32(0x80000000),
                     t ^ jnp.uint32(0x80000000), ~t)
    tval = lax.bitcast_convert_type(tv_u, jnp.float32)
    total = sum_gt + (jnp.float32(_TOPK) - cnt_gt.astype(jnp.float32)) * tval
    mean = total * jnp.float32(1.0 / _TOPK)
    o_ref[...] = jnp.broadcast_to(mean[:, None], (_TC_ROWS, 128))


def kernel(scores):
    sc_out = _sc_topk(scores)
    tc_out = pl.pallas_call(
        _tc_topk_body,
        out_shape=jax.ShapeDtypeStruct((_TC_ROWS, 128), jnp.float32),
        grid=(1,),
        in_specs=[pl.BlockSpec((_TC_ROWS, _N), lambda i: (1, 0))],
        out_specs=pl.BlockSpec((_TC_ROWS, 128), lambda i: (0, 0)),
    )(scores)
    sc_means = sc_out[:, :_RPW].reshape(-1)
    return jnp.concatenate([sc_means, tc_out[:, 0]])
